# baseline (speedup 1.0000x reference)
import functools

import jax
import jax.numpy as jnp
from jax.experimental import pallas as pl
from jax.experimental.pallas import tpu as pltpu


def _router_body(bn, thr, x_ref, w_ref, wt_ref, out_ref, wbuf, anyf):
    p = pl.program_id(0)
    i = pl.program_id(1)
    e = out_ref.shape[1]
    k = wt_ref.shape[0] // e

    @pl.when(p == 0)
    def _compute_weights():
        xb = x_ref[...]
        d = xb.shape[1]
        y = jnp.dot(xb, w_ref[...], preferred_element_type=jnp.float32)
        cols = []
        for pi in range(e // 2):
            proj2 = jnp.dot(y[:, pi * 2 * k:(pi + 1) * 2 * k],
                            wt_ref[pi * 2 * k:(pi + 1) * 2 * k, :],
                            preferred_element_type=jnp.float32)
            for half in range(2):
                proj = proj2[:, half * d:(half + 1) * d]
                d2 = jnp.sum((xb - proj) ** 2, axis=1, keepdims=True)
                cols.append(jnp.sqrt(d2))
        resid = jnp.concatenate(cols, axis=1)
        logits = -resid
        m = jnp.max(logits, axis=1, keepdims=True)
        ex = jnp.exp(logits - m)
        w = ex / jnp.sum(ex, axis=1, keepdims=True)
        wbuf[pl.ds(i * bn, bn), :] = w
        blk_any = jnp.any(w > thr).astype(jnp.int32)
        anyf[0] = jnp.where(i == 0, blk_any, jnp.maximum(anyf[0], blk_any))

    @pl.when(p == 1)
    def _finalize():
        w = wbuf[pl.ds(i * bn, bn), :]
        any_v = anyf[0] != 0
        idx = jax.lax.broadcasted_iota(jnp.int32, w.shape, 1)
        m1 = jnp.max(w, axis=1, keepdims=True)
        i1 = jnp.min(jnp.where(w == m1, idx, e), axis=1, keepdims=True)
        oh1 = idx == i1
        w2 = jnp.where(oh1, -jnp.inf, w)
        m2 = jnp.max(w2, axis=1, keepdims=True)
        i2 = jnp.min(jnp.where(w2 == m2, idx, e), axis=1, keepdims=True)
        tk = oh1 | (idx == i2)
        mask = (((w > thr) & any_v) | (oh1 & jnp.logical_not(any_v))) & tk
        filt = jnp.where(mask, w, 0.0)
        ssum = jnp.sum(filt, axis=1, keepdims=True)
        ssum = jnp.where(ssum == 0.0, 1.0, ssum)
        out_ref[...] = filt / ssum


def kernel(x, x_l, V):
    del x_l
    n, d = x.shape
    e, _, k = V.shape
    ek = e * k
    bn = 512 if n % 512 == 0 else n
    nblk = n // bn
    thr = 1.0 / e

    wmat = jnp.transpose(V, (1, 0, 2)).reshape(d, ek)
    vt = jnp.transpose(V, (0, 2, 1))
    zero = jnp.zeros((e // 2, k, d), jnp.float32)
    top = jnp.concatenate([vt[0::2], zero], axis=2)
    bot = jnp.concatenate([zero, vt[1::2]], axis=2)
    wt = jnp.concatenate([top, bot], axis=1).reshape(ek, 2 * d)

    body = functools.partial(_router_body, bn, thr)
    return pl.pallas_call(
        body,
        grid=(2, nblk),
        in_specs=[
            pl.BlockSpec((bn, d), lambda p, i: (i * (1 - p), 0)),
            pl.BlockSpec((d, ek), lambda p, i: (0, 0)),
            pl.BlockSpec((ek, 2 * d), lambda p, i: (0, 0)),
        ],
        out_specs=pl.BlockSpec((bn, e), lambda p, i: (i, 0)),
        out_shape=jax.ShapeDtypeStruct((n, e), jnp.float32),
        scratch_shapes=[
            pltpu.VMEM((n, e), jnp.float32),
            pltpu.SMEM((1,), jnp.int32),
        ],
        compiler_params=pltpu.CompilerParams(
            vmem_limit_bytes=100 * 1024 * 1024,
        ),
    )(x, wmat, wt)


# speedup vs baseline: 1.0010x; 1.0010x over previous
import functools

import jax
import jax.numpy as jnp
from jax.experimental import pallas as pl
from jax.experimental.pallas import tpu as pltpu


def _weights_body(thr, x_ref, w_ref, wt_ref, wout_ref, flag_ref):
    e = wout_ref.shape[1]
    k = wt_ref.shape[0] // e
    xb = x_ref[...]
    y = jnp.dot(xb, w_ref[...], preferred_element_type=jnp.float32)
    cols = []
    for ei in range(e):
        proj = jnp.dot(y[:, ei * k:(ei + 1) * k],
                       wt_ref[ei * k:(ei + 1) * k, :],
                       preferred_element_type=jnp.float32)
        d2 = jnp.sum((xb - proj) ** 2, axis=1, keepdims=True)
        cols.append(jnp.sqrt(d2))
    resid = jnp.concatenate(cols, axis=1)
    logits = -resid
    m = jnp.max(logits, axis=1, keepdims=True)
    ex = jnp.exp(logits - m)
    w = ex / jnp.sum(ex, axis=1, keepdims=True)
    wout_ref[...] = w
    flag_ref[...] = jnp.broadcast_to(
        jnp.max(jnp.where(w > thr, 1.0, 0.0)), flag_ref.shape)


def _mask_body(thr, w_ref, flags_ref, out_ref):
    e = out_ref.shape[1]
    w = w_ref[...]
    any_v = jnp.max(flags_ref[...]) > 0.0
    idx = jax.lax.broadcasted_iota(jnp.int32, w.shape, 1)
    m1 = jnp.max(w, axis=1, keepdims=True)
    i1 = jnp.min(jnp.where(w == m1, idx, e), axis=1, keepdims=True)
    oh1 = idx == i1
    w2 = jnp.where(oh1, -jnp.inf, w)
    m2 = jnp.max(w2, axis=1, keepdims=True)
    i2 = jnp.min(jnp.where(w2 == m2, idx, e), axis=1, keepdims=True)
    tk = oh1 | (idx == i2)
    mask = (((w > thr) & any_v) | (oh1 & jnp.logical_not(any_v))) & tk
    filt = jnp.where(mask, w, 0.0)
    ssum = jnp.sum(filt, axis=1, keepdims=True)
    ssum = jnp.where(ssum == 0.0, 1.0, ssum)
    out_ref[...] = filt / ssum


def kernel(x, x_l, V):
    del x_l
    n, d = x.shape
    e, _, k = V.shape
    ek = e * k
    bn = 512 if n % 512 == 0 else n
    nblk = n // bn
    thr = 1.0 / e

    wmat = jnp.transpose(V, (1, 0, 2)).reshape(d, ek)
    wt = jnp.transpose(wmat)

    w, flags = pl.pallas_call(
        functools.partial(_weights_body, thr),
        grid=(nblk,),
        in_specs=[
            pl.BlockSpec((bn, d), lambda i: (i, 0)),
            pl.BlockSpec((d, ek), lambda i: (0, 0)),
            pl.BlockSpec((ek, d), lambda i: (0, 0)),
        ],
        out_specs=[
            pl.BlockSpec((bn, e), lambda i: (i, 0)),
            pl.BlockSpec((1, 1, 128), lambda i: (i, 0, 0)),
        ],
        out_shape=[
            jax.ShapeDtypeStruct((n, e), jnp.float32),
            jax.ShapeDtypeStruct((nblk, 1, 128), jnp.float32),
        ],
        compiler_params=pltpu.CompilerParams(
            dimension_semantics=("parallel",),
            vmem_limit_bytes=100 * 1024 * 1024,
        ),
    )(x, wmat, wt)

    return pl.pallas_call(
        functools.partial(_mask_body, thr),
        grid=(nblk,),
        in_specs=[
            pl.BlockSpec((bn, e), lambda i: (i, 0)),
            pl.BlockSpec((nblk, 1, 128), lambda i: (0, 0, 0)),
        ],
        out_specs=pl.BlockSpec((bn, e), lambda i: (i, 0)),
        out_shape=jax.ShapeDtypeStruct((n, e), jnp.float32),
        compiler_params=pltpu.CompilerParams(
            dimension_semantics=("parallel",),
        ),
    )(w, flags)


# speedup vs baseline: 1.0695x; 1.0685x over previous
import functools

import jax
import jax.numpy as jnp
from jax.experimental import pallas as pl
from jax.experimental.pallas import tpu as pltpu


def _router_body(bn, thr, x_ref, w_ref, wt_ref, out_ref, wbuf, anyf):
    p = pl.program_id(0)
    i = pl.program_id(1)
    e = out_ref.shape[1]
    k = wt_ref.shape[0] // e

    @pl.when(p == 0)
    def _compute_weights():
        xb = x_ref[...]
        y = jnp.dot(xb, w_ref[...], preferred_element_type=jnp.float32)
        cols = []
        for ei in range(e):
            proj = jnp.dot(y[:, ei * k:(ei + 1) * k],
                           wt_ref[ei * k:(ei + 1) * k, :],
                           preferred_element_type=jnp.float32)
            d2 = jnp.sum((xb - proj) ** 2, axis=1, keepdims=True)
            cols.append(jnp.sqrt(d2))
        resid = jnp.concatenate(cols, axis=1)
        logits = -resid
        m = jnp.max(logits, axis=1, keepdims=True)
        ex = jnp.exp(logits - m)
        w = ex / jnp.sum(ex, axis=1, keepdims=True)
        wbuf[pl.ds(i * bn, bn), :] = w
        blk_any = jnp.any(w > thr).astype(jnp.int32)
        anyf[0] = jnp.where(i == 0, blk_any, jnp.maximum(anyf[0], blk_any))

    @pl.when(p == 1)
    def _finalize():
        w = wbuf[pl.ds(i * bn, bn), :]
        any_v = anyf[0] != 0
        idx = jax.lax.broadcasted_iota(jnp.int32, w.shape, 1)
        m1 = jnp.max(w, axis=1, keepdims=True)
        i1 = jnp.min(jnp.where(w == m1, idx, e), axis=1, keepdims=True)
        oh1 = idx == i1
        w2 = jnp.where(oh1, -jnp.inf, w)
        m2 = jnp.max(w2, axis=1, keepdims=True)
        i2 = jnp.min(jnp.where(w2 == m2, idx, e), axis=1, keepdims=True)
        tk = oh1 | (idx == i2)
        mask = (((w > thr) & any_v) | (oh1 & jnp.logical_not(any_v))) & tk
        filt = jnp.where(mask, w, 0.0)
        ssum = jnp.sum(filt, axis=1, keepdims=True)
        ssum = jnp.where(ssum == 0.0, 1.0, ssum)
        out_ref[...] = filt / ssum


def kernel(x, x_l, V):
    del x_l
    n, d = x.shape
    e, _, k = V.shape
    ek = e * k
    bn = 1024 if n % 1024 == 0 else n
    nblk = n // bn
    thr = 1.0 / e

    wmat = jnp.transpose(V, (1, 0, 2)).reshape(d, ek)
    wt = jnp.transpose(wmat)

    body = functools.partial(_router_body, bn, thr)
    return pl.pallas_call(
        body,
        grid=(2, nblk),
        in_specs=[
            pl.BlockSpec((bn, d), lambda p, i: (i * (1 - p), 0)),
            pl.BlockSpec((d, ek), lambda p, i: (0, 0)),
            pl.BlockSpec((ek, d), lambda p, i: (0, 0)),
        ],
        out_specs=pl.BlockSpec((bn, e), lambda p, i: (i, 0)),
        out_shape=jax.ShapeDtypeStruct((n, e), jnp.float32),
        scratch_shapes=[
            pltpu.VMEM((n, e), jnp.float32),
            pltpu.SMEM((1,), jnp.int32),
        ],
        compiler_params=pltpu.CompilerParams(
            vmem_limit_bytes=100 * 1024 * 1024,
        ),
    )(x, wmat, wt)
